# Initial kernel scaffold; baseline (speedup 1.0000x reference)
import jax
import jax.numpy as jnp
from jax.experimental import pallas as pl


def kernel(qus, table):
    raise NotImplementedError("write your pallas kernel here")


# speedup vs baseline: 1.1729x; 1.1729x over previous
import functools

import jax
import jax.numpy as jnp
from jax import lax
from jax.experimental import pallas as pl
from jax.experimental.pallas import tpu as pltpu
from jax.experimental.pallas import tpu_sc as plsc

NC = 2
NS = 16
NW = NC * NS

CHUNK = 128


@functools.partial(jax.jit, static_argnames=())
def kernel(qus, table):
    batch, seq = qus.shape
    vocab, dim = table.shape
    n_rows = batch * seq
    assert n_rows % (NW * CHUNK) == 0
    chunks_per_w = n_rows // (NW * CHUNK)

    idx3d = qus.reshape(NW, chunks_per_w, CHUNK).astype(jnp.int32)

    mesh = plsc.VectorSubcoreMesh(core_axis_name="c", subcore_axis_name="s")

    @functools.partial(
        pl.kernel,
        out_type=jax.ShapeDtypeStruct((n_rows, dim), jnp.float32),
        mesh=mesh,
        scratch_types=[
            pltpu.VMEM((chunks_per_w, CHUNK), jnp.int32),
            pltpu.VMEM((CHUNK, dim), jnp.float32),
            pltpu.SemaphoreType.DMA,
        ],
    )
    def emb(idx_hbm, table_hbm, out_hbm, idx_v, rows_v, sem):
        wid = lax.axis_index("s") * NC + lax.axis_index("c")
        base_blk = wid * chunks_per_w
        pltpu.sync_copy(idx_hbm.at[wid], idx_v)
        for j in range(chunks_per_w):
            row0 = pl.multiple_of((base_blk + j) * CHUNK, CHUNK)
            pltpu.async_copy(table_hbm.at[idx_v.at[j]], rows_v, sem).wait()
            pltpu.sync_copy(rows_v, out_hbm.at[pl.ds(row0, CHUNK)])

    out = emb(idx3d, table)
    return out.reshape(batch, seq, dim)


# speedup vs baseline: 1.2972x; 1.1060x over previous
import functools

import jax
import jax.numpy as jnp
from jax import lax
from jax.experimental import pallas as pl
from jax.experimental.pallas import tpu as pltpu
from jax.experimental.pallas import tpu_sc as plsc

NC = 2
NS = 16
NW = NC * NS

CHUNK = 128


@functools.partial(jax.jit, static_argnames=())
def kernel(qus, table):
    batch, seq = qus.shape
    vocab, dim = table.shape
    n_rows = batch * seq
    assert n_rows % (NW * CHUNK) == 0
    chunks_per_w = n_rows // (NW * CHUNK)

    idx3d = qus.reshape(NW, chunks_per_w, CHUNK).astype(jnp.int32)

    mesh = plsc.VectorSubcoreMesh(core_axis_name="c", subcore_axis_name="s")

    NBUF = 4
    LAG = NBUF - 1

    @functools.partial(
        pl.kernel,
        out_type=jax.ShapeDtypeStruct((n_rows, dim), jnp.float32),
        mesh=mesh,
        scratch_types=[
            pltpu.VMEM((chunks_per_w, CHUNK), jnp.int32),
            pltpu.VMEM((NBUF, CHUNK, dim), jnp.float32),
            [pltpu.SemaphoreType.DMA] * NBUF,
            [pltpu.SemaphoreType.DMA] * NBUF,
        ],
    )
    def emb(idx_hbm, table_hbm, out_hbm, idx_v, rows_v, gsems, wsems):
        wid = lax.axis_index("s") * NC + lax.axis_index("c")
        base_blk = wid * chunks_per_w
        pltpu.sync_copy(idx_hbm.at[wid], idx_v)
        gd = [None] * NBUF
        wd = [None] * NBUF
        for j in range(chunks_per_w + LAG):
            if j < chunks_per_w:
                b = j % NBUF
                if wd[b] is not None:
                    wd[b].wait()
                    wd[b] = None
                gd[b] = pltpu.async_copy(
                    table_hbm.at[idx_v.at[j]], rows_v.at[b], gsems[b]
                )
            k = j - LAG
            if k >= 0:
                bk = k % NBUF
                gd[bk].wait()
                row0 = pl.multiple_of((base_blk + k) * CHUNK, CHUNK)
                wd[bk] = pltpu.async_copy(
                    rows_v.at[bk], out_hbm.at[pl.ds(row0, CHUNK)], wsems[bk]
                )
        for b in range(NBUF):
            if wd[b] is not None:
                wd[b].wait()

    out = emb(idx3d, table)
    return out.reshape(batch, seq, dim)


# speedup vs baseline: 1.3137x; 1.0127x over previous
import functools

import jax
import jax.numpy as jnp
from jax import lax
from jax.experimental import pallas as pl
from jax.experimental.pallas import tpu as pltpu
from jax.experimental.pallas import tpu_sc as plsc

NC = 2
NS = 16
NW = NC * NS

CHUNK = 128


@functools.partial(jax.jit, static_argnames=())
def kernel(qus, table):
    batch, seq = qus.shape
    vocab, dim = table.shape
    n_rows = batch * seq
    assert n_rows % (NW * CHUNK) == 0
    chunks_per_w = n_rows // (NW * CHUNK)

    idx3d = qus.reshape(NW, chunks_per_w, CHUNK).astype(jnp.int32)

    mesh = plsc.VectorSubcoreMesh(core_axis_name="c", subcore_axis_name="s")

    NBUF = 6
    LAG = NBUF - 1

    @functools.partial(
        pl.kernel,
        out_type=jax.ShapeDtypeStruct((n_rows, dim), jnp.float32),
        mesh=mesh,
        scratch_types=[
            pltpu.VMEM((chunks_per_w, CHUNK), jnp.int32),
            pltpu.VMEM((NBUF, CHUNK, dim), jnp.float32),
            [pltpu.SemaphoreType.DMA] * NBUF,
            [pltpu.SemaphoreType.DMA] * NBUF,
        ],
    )
    def emb(idx_hbm, table_hbm, out_hbm, idx_v, rows_v, gsems, wsems):
        wid = lax.axis_index("s") * NC + lax.axis_index("c")
        base_blk = wid * chunks_per_w
        pltpu.sync_copy(idx_hbm.at[wid], idx_v)
        gd = [None] * NBUF
        wd = [None] * NBUF
        for j in range(chunks_per_w + LAG):
            if j < chunks_per_w:
                b = j % NBUF
                if wd[b] is not None:
                    wd[b].wait()
                    wd[b] = None
                gd[b] = pltpu.async_copy(
                    table_hbm.at[idx_v.at[j]], rows_v.at[b], gsems[b]
                )
            k = j - LAG
            if k >= 0:
                bk = k % NBUF
                gd[bk].wait()
                row0 = pl.multiple_of((base_blk + k) * CHUNK, CHUNK)
                wd[bk] = pltpu.async_copy(
                    rows_v.at[bk], out_hbm.at[pl.ds(row0, CHUNK)], wsems[bk]
                )
        for b in range(NBUF):
            if wd[b] is not None:
                wd[b].wait()

    out = emb(idx3d, table)
    return out.reshape(batch, seq, dim)


# speedup vs baseline: 2.0077x; 1.5283x over previous
import functools

import jax
import jax.numpy as jnp
from jax import lax
from jax.experimental import pallas as pl
from jax.experimental.pallas import tpu as pltpu
from jax.experimental.pallas import tpu_sc as plsc

NC = 2
NS = 16
NW = NC * NS

CB = 4
NBUF = 4


@jax.jit
def kernel(qus, table):
    batch, seq = qus.shape
    vocab, dim = table.shape
    assert batch % (NW * CB) == 0
    b_per_w = batch // NW
    n_chunks = b_per_w // CB

    idx_in = qus.astype(jnp.int32).reshape(NW, batch // (NW * CB), CB * seq)

    mesh = plsc.VectorSubcoreMesh(core_axis_name="c", subcore_axis_name="s")
    LAG = NBUF - 1

    @functools.partial(
        pl.kernel,
        out_type=jax.ShapeDtypeStruct((batch, seq, dim), jnp.float32),
        mesh=mesh,
        scratch_types=[
            pltpu.VMEM((n_chunks, CB * seq), jnp.int32),
            pltpu.VMEM((NBUF, CB * seq, dim), jnp.float32),
            [pltpu.SemaphoreType.DMA] * NBUF,
            [pltpu.SemaphoreType.DMA] * NBUF,
        ],
    )
    def emb(idx_hbm, table_hbm, out_hbm, idx_v, rows_v, gsems, wsems):
        wid = lax.axis_index("s") * NC + lax.axis_index("c")
        base_b = pl.multiple_of(wid * b_per_w, b_per_w)
        pltpu.sync_copy(idx_hbm.at[wid], idx_v)
        gd = [None] * NBUF
        wd = [None] * NBUF
        for j in range(n_chunks + LAG):
            if j < n_chunks:
                b = j % NBUF
                if wd[b] is not None:
                    wd[b].wait()
                    wd[b] = None
                gd[b] = pltpu.async_copy(
                    table_hbm.at[idx_v.at[j]],
                    rows_v.at[b],
                    gsems[b],
                )
            k = j - LAG
            if k >= 0:
                bk = k % NBUF
                gd[bk].wait()
                b0 = pl.multiple_of(base_b + k * CB, CB)
                wd[bk] = pltpu.async_copy(
                    rows_v.at[bk].reshape(CB, seq, dim),
                    out_hbm.at[pl.ds(b0, CB)],
                    wsems[bk],
                )
        for b in range(NBUF):
            if wd[b] is not None:
                wd[b].wait()

    return emb(idx_in, table)


# speedup vs baseline: 2.0116x; 1.0019x over previous
import functools

import jax
import jax.numpy as jnp
from jax import lax
from jax.experimental import pallas as pl
from jax.experimental.pallas import tpu as pltpu
from jax.experimental.pallas import tpu_sc as plsc

NC = 2
NS = 16
NW = NC * NS

CB = 4
NBUF = 4


@jax.jit
def kernel(qus, table):
    batch, seq = qus.shape
    vocab, dim = table.shape
    assert batch % (NW * CB) == 0
    b_per_w = batch // NW
    n_chunks = b_per_w // CB

    idx_in = qus.astype(jnp.int32).reshape(NW, batch // (NW * CB), CB * seq)

    mesh = plsc.VectorSubcoreMesh(core_axis_name="c", subcore_axis_name="s")
    LAG = NBUF - 1

    @functools.partial(
        pl.kernel,
        out_type=jax.ShapeDtypeStruct((batch, seq, dim), jnp.float32),
        mesh=mesh,
        scratch_types=[
            pltpu.VMEM((n_chunks, CB * seq), jnp.int32),
            pltpu.VMEM((NBUF, CB * seq, dim), jnp.float32),
            [pltpu.SemaphoreType.DMA] * NBUF,
            [pltpu.SemaphoreType.DMA] * NBUF,
        ],
        compiler_params=pltpu.CompilerParams(use_tc_tiling_on_sc=True),
    )
    def emb(idx_hbm, table_hbm, out_hbm, idx_v, rows_v, gsems, wsems):
        wid = lax.axis_index("s") * NC + lax.axis_index("c")
        base_b = pl.multiple_of(wid * b_per_w, b_per_w)
        pltpu.sync_copy(idx_hbm.at[wid], idx_v)
        gd = [None] * NBUF
        wd = [None] * NBUF
        for j in range(n_chunks + LAG):
            if j < n_chunks:
                b = j % NBUF
                if wd[b] is not None:
                    wd[b].wait()
                    wd[b] = None
                gd[b] = pltpu.async_copy(
                    table_hbm.at[idx_v.at[j]],
                    rows_v.at[b],
                    gsems[b],
                )
            k = j - LAG
            if k >= 0:
                bk = k % NBUF
                gd[bk].wait()
                b0 = pl.multiple_of(base_b + k * CB, CB)
                wd[bk] = pltpu.async_copy(
                    rows_v.at[bk].reshape(CB, seq, dim),
                    out_hbm.at[pl.ds(b0, CB)],
                    wsems[bk],
                )
        for b in range(NBUF):
            if wd[b] is not None:
                wd[b].wait()

    return emb(idx_in, table)
